# baseline (speedup 1.0000x reference)
import jax
import jax.numpy as jnp
from jax import lax
from jax.experimental import pallas as pl
from jax.experimental.pallas import tpu as pltpu

LEVEL_SIZES_K = [6, 24, 96, 384, 1536, 6144]
NUM_LABELS_K = 8190
IN_FEAT = 768
K_AUG = 2944
N_PAD = 8192
SEG_OFF = [0, 768, 774, 798, 894, 1278]
SEG_W = [768, 6, 24, 96, 384, 1536]
ONES_COL = 2814
Y0_OFF = 2815


def _phase_a_kernel(x_ref, w0t, b0, w1t, b1, w2t, b2, w3t, b3, w4t, b4,
                    out_ref, cur_ref):
    bb = x_ref.shape[0]
    bf16 = jnp.bfloat16
    out_ref[...] = jnp.zeros((bb, K_AUG), bf16)
    cur_ref[...] = jnp.zeros(cur_ref.shape, jnp.float32)
    x = x_ref[...]
    rx = jnp.maximum(x, 0.0)
    cur_ref[:, 0:768] = rx
    out_ref[:, 0:768] = rx.astype(bf16)
    y0 = jnp.dot(x, w0t[...], preferred_element_type=jnp.float32) + b0[...]
    ry0 = jnp.maximum(y0, 0.0)
    cur_ref[:, 768:774] = ry0
    out_ref[:, 768:774] = ry0.astype(bf16)
    out_ref[:, Y0_OFF:Y0_OFF + 6] = y0.astype(bf16)
    out_ref[:, ONES_COL:ONES_COL + 1] = jnp.ones((bb, 1), bf16)
    for lvl, (wt, b) in enumerate(((w1t, b1), (w2t, b2), (w3t, b3),
                                   (w4t, b4)), start=1):
        k_in = wt.shape[0]
        y = jnp.dot(cur_ref[:, 0:k_in], wt[...],
                    preferred_element_type=jnp.float32) + b[...]
        o = SEG_OFF[lvl + 1]
        ry = jnp.maximum(y, 0.0)
        if o + SEG_W[lvl + 1] <= cur_ref.shape[1]:
            cur_ref[:, o:o + SEG_W[lvl + 1]] = ry
        out_ref[:, o:o + SEG_W[lvl + 1]] = ry.astype(bf16)


def _gather_kernel(idx_ref, wsrc_ref, out_ref, stage, stage16, sem_in,
                   sem_out):
    step = pl.program_id(0)
    rows = stage.shape[0]
    base = step * rows

    def issue(i, _):
        src = idx_ref[base + i]
        pltpu.make_async_copy(
            wsrc_ref.at[src], stage.at[i], sem_in).start()
        return 0

    lax.fori_loop(0, rows, issue, 0)
    pltpu.make_async_copy(
        wsrc_ref.at[pl.ds(0, rows)], stage.at[pl.ds(0, rows)], sem_in,
    ).wait()
    stage16[...] = stage[...].astype(jnp.bfloat16)
    out_cp = pltpu.make_async_copy(
        stage16, out_ref.at[pl.ds(base, rows)], sem_out)
    out_cp.start()
    out_cp.wait()


def _matmul_kernel(cur_ref, w_ref, out_ref):
    out_ref[...] = lax.dot_general(
        cur_ref[...], w_ref[...],
        dimension_numbers=(((1,), (1,)), ((), ())),
        preferred_element_type=jnp.float32)


def kernel(x, W0, b0, W1, b1, W2, b2, W3, b3, W4, b4, W5, b5, labels):
    batch = x.shape[0]
    f32 = jnp.float32
    Ws = [W0, W1, W2, W3, W4, W5]
    bs = [b0, b1, b2, b3, b4, b5]

    wts = [Ws[lvl].T for lvl in range(5)]
    brs = [b.reshape(1, -1) for b in bs]

    blocks = []
    lvl0 = jnp.concatenate([
        jnp.zeros((6, Y0_OFF), f32),
        jnp.eye(6, dtype=f32),
        jnp.zeros((6, K_AUG - Y0_OFF - 6), f32),
    ], axis=1)
    blocks.append(lvl0)
    for lvl in range(1, 6):
        w = Ws[lvl]
        n = w.shape[0]
        blocks.append(jnp.concatenate([
            w,
            jnp.zeros((n, ONES_COL - w.shape[1]), f32),
            bs[lvl].reshape(n, 1),
            jnp.zeros((n, K_AUG - ONES_COL - 1), f32),
        ], axis=1))
    blocks.append(jnp.zeros((2, K_AUG), f32))
    wsrc = jnp.concatenate(blocks, axis=0)

    labels_i = labels.astype(jnp.int32)
    inv = jnp.argsort(labels_i).astype(jnp.int32)
    inv_ext = jnp.concatenate(
        [inv, jnp.array([NUM_LABELS_K, NUM_LABELS_K + 1], jnp.int32)])

    bb = 256
    grid1 = (batch // bb,)
    cur = pl.pallas_call(
        _phase_a_kernel,
        grid=grid1,
        in_specs=[pl.BlockSpec((bb, IN_FEAT), lambda i: (i, 0))] + [
            spec for lvl in range(5) for spec in (
                pl.BlockSpec(wts[lvl].shape, lambda i: (0, 0)),
                pl.BlockSpec(brs[lvl].shape, lambda i: (0, 0)),
            )
        ],
        out_specs=pl.BlockSpec((bb, K_AUG), lambda i: (i, 0)),
        out_shape=jax.ShapeDtypeStruct((batch, K_AUG), jnp.bfloat16),
        scratch_shapes=[pltpu.VMEM((bb, SEG_OFF[5]), f32)],
        compiler_params=pltpu.CompilerParams(
            dimension_semantics=("parallel",),
            vmem_limit_bytes=56 * 1024 * 1024,
        ),
    )(x, wts[0], brs[0], wts[1], brs[1], wts[2], brs[2], wts[3], brs[3],
      wts[4], brs[4])

    rows_blk = 256
    wsrc3 = wsrc.reshape(N_PAD, 1, K_AUG)
    wbig3 = pl.pallas_call(
        _gather_kernel,
        grid=(N_PAD // rows_blk,),
        in_specs=[
            pl.BlockSpec(memory_space=pltpu.SMEM),
            pl.BlockSpec(memory_space=pl.ANY),
        ],
        out_specs=pl.BlockSpec(memory_space=pl.ANY),
        out_shape=jax.ShapeDtypeStruct((N_PAD, 1, K_AUG), jnp.bfloat16),
        scratch_shapes=[
            pltpu.VMEM((rows_blk, 1, K_AUG), f32),
            pltpu.VMEM((rows_blk, 1, K_AUG), jnp.bfloat16),
            pltpu.SemaphoreType.DMA,
            pltpu.SemaphoreType.DMA,
        ],
        compiler_params=pltpu.CompilerParams(
            dimension_semantics=("parallel",),
        ),
    )(inv_ext, wsrc3)

    wbig16 = wbig3.reshape(N_PAD, K_AUG)
    bm, bn = 512, 2048
    grid3 = (N_PAD // bn, batch // bm)
    logits = pl.pallas_call(
        _matmul_kernel,
        grid=grid3,
        in_specs=[
            pl.BlockSpec((bm, K_AUG), lambda c, b: (b, 0)),
            pl.BlockSpec((bn, K_AUG), lambda c, b: (c, 0)),
        ],
        out_specs=pl.BlockSpec((bm, bn), lambda c, b: (b, c)),
        out_shape=jax.ShapeDtypeStruct((batch, NUM_LABELS_K), f32),
        compiler_params=pltpu.CompilerParams(
            dimension_semantics=("parallel", "arbitrary"),
            vmem_limit_bytes=56 * 1024 * 1024,
        ),
    )(cur, wbig16)

    return logits


# speedup vs baseline: 1.0548x; 1.0548x over previous
import jax
import jax.numpy as jnp
from jax import lax
from jax.experimental import pallas as pl
from jax.experimental.pallas import tpu as pltpu

LEVEL_SIZES_K = [6, 24, 96, 384, 1536, 6144]
NUM_LABELS_K = 8190
IN_FEAT = 768
K_AUG = 2944
N_PAD = 8192
SEG_OFF = [0, 768, 774, 798, 894, 1278]
SEG_W = [768, 6, 24, 96, 384, 1536]
ONES_COL = 2814
Y0_OFF = 2815


def _phase_a_kernel(x_ref, w0t, b0, w1t, b1, w2t, b2, w3t, b3, w4t, b4,
                    out_ref, cur_ref):
    bb = x_ref.shape[0]
    bf16 = jnp.bfloat16
    out_ref[...] = jnp.zeros((bb, K_AUG), bf16)
    cur_ref[...] = jnp.zeros(cur_ref.shape, jnp.float32)
    x = x_ref[...]
    rx = jnp.maximum(x, 0.0)
    cur_ref[:, 0:768] = rx
    out_ref[:, 0:768] = rx.astype(bf16)
    y0 = jnp.dot(x, w0t[...], preferred_element_type=jnp.float32) + b0[...]
    ry0 = jnp.maximum(y0, 0.0)
    cur_ref[:, 768:774] = ry0
    out_ref[:, 768:774] = ry0.astype(bf16)
    out_ref[:, Y0_OFF:Y0_OFF + 6] = y0.astype(bf16)
    out_ref[:, ONES_COL:ONES_COL + 1] = jnp.ones((bb, 1), bf16)
    for lvl, (wt, b) in enumerate(((w1t, b1), (w2t, b2), (w3t, b3),
                                   (w4t, b4)), start=1):
        k_in = wt.shape[0]
        y = jnp.dot(cur_ref[:, 0:k_in], wt[...],
                    preferred_element_type=jnp.float32) + b[...]
        o = SEG_OFF[lvl + 1]
        ry = jnp.maximum(y, 0.0)
        if o + SEG_W[lvl + 1] <= cur_ref.shape[1]:
            cur_ref[:, o:o + SEG_W[lvl + 1]] = ry
        out_ref[:, o:o + SEG_W[lvl + 1]] = ry.astype(bf16)


def _gather_kernel(idx_ref, wsrc_ref, out_ref, stage, sem_in, sem_out):
    step = pl.program_id(0)
    rows = stage.shape[0]
    base = step * rows

    def issue(i, _):
        src = idx_ref[base + i]
        pltpu.make_async_copy(
            wsrc_ref.at[src], stage.at[i], sem_in).start()
        return 0

    lax.fori_loop(0, rows, issue, 0)
    pltpu.make_async_copy(
        wsrc_ref.at[pl.ds(0, rows)], stage.at[pl.ds(0, rows)], sem_in,
    ).wait()
    out_cp = pltpu.make_async_copy(
        stage, out_ref.at[pl.ds(base, rows)], sem_out)
    out_cp.start()
    out_cp.wait()


def _matmul_kernel(cur_ref, w_ref, out_ref):
    out_ref[...] = lax.dot_general(
        cur_ref[...], w_ref[...],
        dimension_numbers=(((1,), (1,)), ((), ())),
        preferred_element_type=jnp.float32)


def kernel(x, W0, b0, W1, b1, W2, b2, W3, b3, W4, b4, W5, b5, labels):
    batch = x.shape[0]
    f32 = jnp.float32
    Ws = [W0, W1, W2, W3, W4, W5]
    bs = [b0, b1, b2, b3, b4, b5]

    wts = [Ws[lvl].T for lvl in range(5)]
    brs = [b.reshape(1, -1) for b in bs]

    blocks = []
    lvl0 = jnp.concatenate([
        jnp.zeros((6, Y0_OFF), f32),
        jnp.eye(6, dtype=f32),
        jnp.zeros((6, K_AUG - Y0_OFF - 6), f32),
    ], axis=1)
    blocks.append(lvl0)
    for lvl in range(1, 6):
        w = Ws[lvl]
        n = w.shape[0]
        blocks.append(jnp.concatenate([
            w,
            jnp.zeros((n, ONES_COL - w.shape[1]), f32),
            bs[lvl].reshape(n, 1),
            jnp.zeros((n, K_AUG - ONES_COL - 1), f32),
        ], axis=1))
    blocks.append(jnp.zeros((2, K_AUG), f32))
    wsrc = jnp.concatenate(blocks, axis=0)

    labels_i = labels.astype(jnp.int32)
    inv = jnp.argsort(labels_i).astype(jnp.int32)
    inv_ext = jnp.concatenate(
        [inv, jnp.array([NUM_LABELS_K, NUM_LABELS_K + 1], jnp.int32)])

    bb = 256
    grid1 = (batch // bb,)
    cur = pl.pallas_call(
        _phase_a_kernel,
        grid=grid1,
        in_specs=[pl.BlockSpec((bb, IN_FEAT), lambda i: (i, 0))] + [
            spec for lvl in range(5) for spec in (
                pl.BlockSpec(wts[lvl].shape, lambda i: (0, 0)),
                pl.BlockSpec(brs[lvl].shape, lambda i: (0, 0)),
            )
        ],
        out_specs=pl.BlockSpec((bb, K_AUG), lambda i: (i, 0)),
        out_shape=jax.ShapeDtypeStruct((batch, K_AUG), jnp.bfloat16),
        scratch_shapes=[pltpu.VMEM((bb, SEG_OFF[5]), f32)],
        compiler_params=pltpu.CompilerParams(
            dimension_semantics=("parallel",),
            vmem_limit_bytes=56 * 1024 * 1024,
        ),
    )(x, wts[0], brs[0], wts[1], brs[1], wts[2], brs[2], wts[3], brs[3],
      wts[4], brs[4])

    rows_blk = 256
    wsrc3 = wsrc.reshape(N_PAD, 1, K_AUG)
    wbig3 = pl.pallas_call(
        _gather_kernel,
        grid=(N_PAD // rows_blk,),
        in_specs=[
            pl.BlockSpec(memory_space=pltpu.SMEM),
            pl.BlockSpec(memory_space=pl.ANY),
        ],
        out_specs=pl.BlockSpec(memory_space=pl.ANY),
        out_shape=jax.ShapeDtypeStruct((N_PAD, 1, K_AUG), f32),
        scratch_shapes=[
            pltpu.VMEM((rows_blk, 1, K_AUG), f32),
            pltpu.SemaphoreType.DMA,
            pltpu.SemaphoreType.DMA,
        ],
        compiler_params=pltpu.CompilerParams(
            dimension_semantics=("parallel",),
        ),
    )(inv_ext, wsrc3)

    wbig16 = wbig3.reshape(N_PAD, K_AUG).astype(jnp.bfloat16)
    bm, bn = 1024, 2048
    grid3 = (N_PAD // bn, batch // bm)
    logits = pl.pallas_call(
        _matmul_kernel,
        grid=grid3,
        in_specs=[
            pl.BlockSpec((bm, K_AUG), lambda c, b: (b, 0)),
            pl.BlockSpec((bn, K_AUG), lambda c, b: (c, 0)),
        ],
        out_specs=pl.BlockSpec((bm, bn), lambda c, b: (b, c)),
        out_shape=jax.ShapeDtypeStruct((batch, NUM_LABELS_K), f32),
        compiler_params=pltpu.CompilerParams(
            dimension_semantics=("parallel", "arbitrary"),
            vmem_limit_bytes=56 * 1024 * 1024,
        ),
    )(cur, wbig16)

    return logits


# speedup vs baseline: 1.0626x; 1.0074x over previous
import jax
import jax.numpy as jnp
from jax import lax
from jax.experimental import pallas as pl
from jax.experimental.pallas import tpu as pltpu

LEVEL_SIZES_K = [6, 24, 96, 384, 1536, 6144]
NUM_LABELS_K = 8190
IN_FEAT = 768
K_AUG = 2944
N_PAD = 8192
SEG_OFF = [0, 768, 774, 798, 894, 1278]
SEG_W = [768, 6, 24, 96, 384, 1536]
ONES_COL = 2814
Y0_OFF = 2815


def _phase_a_kernel(x_ref, w0t, b0, w1t, b1, w2t, b2, w3t, b3, w4t, b4,
                    out_ref, cur_ref):
    bb = x_ref.shape[0]
    bf16 = jnp.bfloat16
    out_ref[...] = jnp.zeros((bb, K_AUG), bf16)
    cur_ref[...] = jnp.zeros(cur_ref.shape, jnp.float32)
    x = x_ref[...]
    rx = jnp.maximum(x, 0.0)
    cur_ref[:, 0:768] = rx
    out_ref[:, 0:768] = rx.astype(bf16)
    y0 = jnp.dot(x, w0t[...], preferred_element_type=jnp.float32) + b0[...]
    ry0 = jnp.maximum(y0, 0.0)
    cur_ref[:, 768:774] = ry0
    out_ref[:, 768:774] = ry0.astype(bf16)
    out_ref[:, Y0_OFF:Y0_OFF + 6] = y0.astype(bf16)
    out_ref[:, ONES_COL:ONES_COL + 1] = jnp.ones((bb, 1), bf16)
    for lvl, (wt, b) in enumerate(((w1t, b1), (w2t, b2), (w3t, b3),
                                   (w4t, b4)), start=1):
        k_in = wt.shape[0]
        y = jnp.dot(cur_ref[:, 0:k_in], wt[...],
                    preferred_element_type=jnp.float32) + b[...]
        o = SEG_OFF[lvl + 1]
        ry = jnp.maximum(y, 0.0)
        if o + SEG_W[lvl + 1] <= cur_ref.shape[1]:
            cur_ref[:, o:o + SEG_W[lvl + 1]] = ry
        out_ref[:, o:o + SEG_W[lvl + 1]] = ry.astype(bf16)


def _scatter_kernel(lab_ref, wsrc_ref, out_ref, stage, sem_in, sem_out):
    step = pl.program_id(0)
    rows = stage.shape[0]
    base = step * rows

    in_cp = pltpu.make_async_copy(
        wsrc_ref.at[pl.ds(base, rows)], stage, sem_in)
    in_cp.start()
    in_cp.wait()

    def issue(i, _):
        dst = lab_ref[base + i]
        pltpu.make_async_copy(
            stage.at[i], out_ref.at[dst], sem_out).start()
        return 0

    lax.fori_loop(0, rows, issue, 0)
    pltpu.make_async_copy(
        stage.at[pl.ds(0, rows)], out_ref.at[pl.ds(0, rows)], sem_out,
    ).wait()


def _matmul_kernel(cur_ref, w_ref, out_ref):
    out_ref[...] = lax.dot_general(
        cur_ref[...], w_ref[...],
        dimension_numbers=(((1,), (1,)), ((), ())),
        preferred_element_type=jnp.float32)


def kernel(x, W0, b0, W1, b1, W2, b2, W3, b3, W4, b4, W5, b5, labels):
    batch = x.shape[0]
    f32 = jnp.float32
    Ws = [W0, W1, W2, W3, W4, W5]
    bs = [b0, b1, b2, b3, b4, b5]

    wts = [Ws[lvl].T for lvl in range(5)]
    brs = [b.reshape(1, -1) for b in bs]

    blocks = []
    lvl0 = jnp.concatenate([
        jnp.zeros((6, Y0_OFF), f32),
        jnp.eye(6, dtype=f32),
        jnp.zeros((6, K_AUG - Y0_OFF - 6), f32),
    ], axis=1)
    blocks.append(lvl0)
    for lvl in range(1, 6):
        w = Ws[lvl]
        n = w.shape[0]
        blocks.append(jnp.concatenate([
            w,
            jnp.zeros((n, ONES_COL - w.shape[1]), f32),
            bs[lvl].reshape(n, 1),
            jnp.zeros((n, K_AUG - ONES_COL - 1), f32),
        ], axis=1))
    blocks.append(jnp.zeros((2, K_AUG), f32))
    wsrc = jnp.concatenate(blocks, axis=0)

    labels_i = labels.astype(jnp.int32)
    lab_ext = jnp.concatenate(
        [labels_i, jnp.array([NUM_LABELS_K, NUM_LABELS_K + 1], jnp.int32)])

    bb = 256
    grid1 = (batch // bb,)
    cur = pl.pallas_call(
        _phase_a_kernel,
        grid=grid1,
        in_specs=[pl.BlockSpec((bb, IN_FEAT), lambda i: (i, 0))] + [
            spec for lvl in range(5) for spec in (
                pl.BlockSpec(wts[lvl].shape, lambda i: (0, 0)),
                pl.BlockSpec(brs[lvl].shape, lambda i: (0, 0)),
            )
        ],
        out_specs=pl.BlockSpec((bb, K_AUG), lambda i: (i, 0)),
        out_shape=jax.ShapeDtypeStruct((batch, K_AUG), jnp.bfloat16),
        scratch_shapes=[pltpu.VMEM((bb, SEG_OFF[5]), f32)],
        compiler_params=pltpu.CompilerParams(
            dimension_semantics=("parallel",),
            vmem_limit_bytes=56 * 1024 * 1024,
        ),
    )(x, wts[0], brs[0], wts[1], brs[1], wts[2], brs[2], wts[3], brs[3],
      wts[4], brs[4])

    rows_blk = 256
    wsrc3 = wsrc.reshape(N_PAD, 1, K_AUG)
    wbig3 = pl.pallas_call(
        _scatter_kernel,
        grid=(N_PAD // rows_blk,),
        in_specs=[
            pl.BlockSpec(memory_space=pltpu.SMEM),
            pl.BlockSpec(memory_space=pl.ANY),
        ],
        out_specs=pl.BlockSpec(memory_space=pl.ANY),
        out_shape=jax.ShapeDtypeStruct((N_PAD, 1, K_AUG), f32),
        scratch_shapes=[
            pltpu.VMEM((rows_blk, 1, K_AUG), f32),
            pltpu.SemaphoreType.DMA,
            pltpu.SemaphoreType.DMA,
        ],
        compiler_params=pltpu.CompilerParams(
            dimension_semantics=("parallel",),
        ),
    )(lab_ext, wsrc3)

    wbig16 = wbig3.reshape(N_PAD, K_AUG).astype(jnp.bfloat16)
    bm, bn = 1024, 2048
    grid3 = (N_PAD // bn, batch // bm)
    logits = pl.pallas_call(
        _matmul_kernel,
        grid=grid3,
        in_specs=[
            pl.BlockSpec((bm, K_AUG), lambda c, b: (b, 0)),
            pl.BlockSpec((bn, K_AUG), lambda c, b: (c, 0)),
        ],
        out_specs=pl.BlockSpec((bm, bn), lambda c, b: (b, c)),
        out_shape=jax.ShapeDtypeStruct((batch, NUM_LABELS_K), f32),
        compiler_params=pltpu.CompilerParams(
            dimension_semantics=("parallel", "arbitrary"),
            vmem_limit_bytes=56 * 1024 * 1024,
        ),
    )(cur, wbig16)

    return logits


# speedup vs baseline: 1.0872x; 1.0231x over previous
import jax
import jax.numpy as jnp
from jax import lax
from jax.experimental import pallas as pl
from jax.experimental.pallas import tpu as pltpu

LEVEL_SIZES_K = [6, 24, 96, 384, 1536, 6144]
NUM_LABELS_K = 8190
IN_FEAT = 768
K_AUG = 2944
N_PAD = 8192
SEG_OFF = [0, 768, 774, 798, 894, 1278]
SEG_W = [768, 6, 24, 96, 384, 1536]
ONES_COL = 2814
Y0_OFF = 2815


def _phase_a_kernel(x_ref, w0t, b0, w1t, b1, w2t, b2, w3t, b3, w4t, b4,
                    out_ref, cur_ref):
    bb = x_ref.shape[0]
    bf16 = jnp.bfloat16
    out_ref[...] = jnp.zeros((bb, K_AUG), bf16)
    cur_ref[...] = jnp.zeros(cur_ref.shape, jnp.float32)
    x = x_ref[...]
    rx = jnp.maximum(x, 0.0)
    cur_ref[:, 0:768] = rx
    out_ref[:, 0:768] = rx.astype(bf16)
    y0 = jnp.dot(x, w0t[...], preferred_element_type=jnp.float32) + b0[...]
    ry0 = jnp.maximum(y0, 0.0)
    cur_ref[:, 768:774] = ry0
    out_ref[:, 768:774] = ry0.astype(bf16)
    out_ref[:, Y0_OFF:Y0_OFF + 6] = y0.astype(bf16)
    out_ref[:, ONES_COL:ONES_COL + 1] = jnp.ones((bb, 1), bf16)
    for lvl, (wt, b) in enumerate(((w1t, b1), (w2t, b2), (w3t, b3),
                                   (w4t, b4)), start=1):
        k_in = wt.shape[0]
        y = jnp.dot(cur_ref[:, 0:k_in], wt[...],
                    preferred_element_type=jnp.float32) + b[...]
        o = SEG_OFF[lvl + 1]
        ry = jnp.maximum(y, 0.0)
        if o + SEG_W[lvl + 1] <= cur_ref.shape[1]:
            cur_ref[:, o:o + SEG_W[lvl + 1]] = ry
        out_ref[:, o:o + SEG_W[lvl + 1]] = ry.astype(bf16)


def _scatter_kernel(lab_ref, wsrc_ref, out_ref, stage, sem_in, sem_out):
    step = pl.program_id(0)
    rows = stage.shape[0]
    base = step * rows

    in_cp = pltpu.make_async_copy(
        wsrc_ref.at[pl.ds(base, rows)], stage, sem_in)
    in_cp.start()
    in_cp.wait()

    def issue(i, _):
        dst = lab_ref[base + i]
        pltpu.make_async_copy(
            stage.at[i], out_ref.at[dst], sem_out).start()
        return 0

    lax.fori_loop(0, rows, issue, 0)
    pltpu.make_async_copy(
        stage.at[pl.ds(0, rows)], out_ref.at[pl.ds(0, rows)], sem_out,
    ).wait()


def _matmul_kernel(cur_ref, w_ref, out_ref):
    out_ref[...] = lax.dot_general(
        cur_ref[...], w_ref[...],
        dimension_numbers=(((1,), (1,)), ((), ())),
        preferred_element_type=jnp.float32)


def kernel(x, W0, b0, W1, b1, W2, b2, W3, b3, W4, b4, W5, b5, labels):
    batch = x.shape[0]
    f32 = jnp.float32
    Ws = [W0, W1, W2, W3, W4, W5]
    bs = [b0, b1, b2, b3, b4, b5]

    wts = [Ws[lvl].T for lvl in range(5)]
    brs = [b.reshape(1, -1) for b in bs]

    blocks = []
    lvl0 = jnp.concatenate([
        jnp.zeros((6, Y0_OFF), f32),
        jnp.eye(6, dtype=f32),
        jnp.zeros((6, K_AUG - Y0_OFF - 6), f32),
    ], axis=1)
    blocks.append(lvl0)
    for lvl in range(1, 6):
        w = Ws[lvl]
        n = w.shape[0]
        blocks.append(jnp.concatenate([
            w,
            jnp.zeros((n, ONES_COL - w.shape[1]), f32),
            bs[lvl].reshape(n, 1),
            jnp.zeros((n, K_AUG - ONES_COL - 1), f32),
        ], axis=1))
    blocks.append(jnp.zeros((2, K_AUG), f32))
    wsrc = jnp.concatenate(blocks, axis=0)

    labels_i = labels.astype(jnp.int32)
    lab_ext = jnp.concatenate(
        [labels_i, jnp.array([NUM_LABELS_K, NUM_LABELS_K + 1], jnp.int32)])

    bb = 256
    grid1 = (batch // bb,)
    cur = pl.pallas_call(
        _phase_a_kernel,
        grid=grid1,
        in_specs=[pl.BlockSpec((bb, IN_FEAT), lambda i: (i, 0))] + [
            spec for lvl in range(5) for spec in (
                pl.BlockSpec(wts[lvl].shape, lambda i: (0, 0)),
                pl.BlockSpec(brs[lvl].shape, lambda i: (0, 0)),
            )
        ],
        out_specs=pl.BlockSpec((bb, K_AUG), lambda i: (i, 0)),
        out_shape=jax.ShapeDtypeStruct((batch, K_AUG), jnp.bfloat16),
        scratch_shapes=[pltpu.VMEM((bb, SEG_OFF[5]), f32)],
        compiler_params=pltpu.CompilerParams(
            dimension_semantics=("parallel",),
            vmem_limit_bytes=56 * 1024 * 1024,
        ),
    )(x, wts[0], brs[0], wts[1], brs[1], wts[2], brs[2], wts[3], brs[3],
      wts[4], brs[4])

    rows_blk = 512
    wsrc3 = wsrc.reshape(N_PAD, 1, K_AUG)
    wbig3 = pl.pallas_call(
        _scatter_kernel,
        grid=(N_PAD // rows_blk,),
        in_specs=[
            pl.BlockSpec(memory_space=pltpu.SMEM),
            pl.BlockSpec(memory_space=pl.ANY),
        ],
        out_specs=pl.BlockSpec(memory_space=pl.ANY),
        out_shape=jax.ShapeDtypeStruct((N_PAD, 1, K_AUG), f32),
        scratch_shapes=[
            pltpu.VMEM((rows_blk, 1, K_AUG), f32),
            pltpu.SemaphoreType.DMA,
            pltpu.SemaphoreType.DMA,
        ],
        compiler_params=pltpu.CompilerParams(
            dimension_semantics=("parallel",),
        ),
    )(lab_ext, wsrc3)

    wbig16 = wbig3.reshape(N_PAD, K_AUG).astype(jnp.bfloat16)
    bm, bn = 1024, 2048
    grid3 = (N_PAD // bn, batch // bm)
    logits = pl.pallas_call(
        _matmul_kernel,
        grid=grid3,
        in_specs=[
            pl.BlockSpec((bm, K_AUG), lambda c, b: (b, 0)),
            pl.BlockSpec((bn, K_AUG), lambda c, b: (c, 0)),
        ],
        out_specs=pl.BlockSpec((bm, bn), lambda c, b: (b, c)),
        out_shape=jax.ShapeDtypeStruct((batch, NUM_LABELS_K), f32),
        compiler_params=pltpu.CompilerParams(
            dimension_semantics=("parallel", "arbitrary"),
            vmem_limit_bytes=56 * 1024 * 1024,
        ),
    )(cur, wbig16)

    return logits


# speedup vs baseline: 1.1061x; 1.0174x over previous
import jax
import jax.numpy as jnp
from jax import lax
from jax.experimental import pallas as pl
from jax.experimental.pallas import tpu as pltpu

LEVEL_SIZES_K = [6, 24, 96, 384, 1536, 6144]
NUM_LABELS_K = 8190
IN_FEAT = 768
K_AUG = 2944
N_PAD = 8192
SEG_OFF = [0, 768, 774, 798, 894, 1278]
SEG_W = [768, 6, 24, 96, 384, 1536]
ONES_COL = 2814
Y0_OFF = 2815


def _phase_a_kernel(x_ref, w0t, b0, w1t, b1, w2t, b2, w3t, b3, w4t, b4,
                    out_ref, cur_ref):
    bb = x_ref.shape[0]
    bf16 = jnp.bfloat16
    out_ref[...] = jnp.zeros((bb, K_AUG), bf16)
    cur_ref[...] = jnp.zeros(cur_ref.shape, jnp.float32)
    x = x_ref[...]
    rx = jnp.maximum(x, 0.0)
    cur_ref[:, 0:768] = rx
    out_ref[:, 0:768] = rx.astype(bf16)
    y0 = jnp.dot(x, w0t[...], preferred_element_type=jnp.float32) + b0[...]
    ry0 = jnp.maximum(y0, 0.0)
    cur_ref[:, 768:774] = ry0
    out_ref[:, 768:774] = ry0.astype(bf16)
    out_ref[:, Y0_OFF:Y0_OFF + 6] = y0.astype(bf16)
    out_ref[:, ONES_COL:ONES_COL + 1] = jnp.ones((bb, 1), bf16)
    for lvl, (wt, b) in enumerate(((w1t, b1), (w2t, b2), (w3t, b3),
                                   (w4t, b4)), start=1):
        k_in = wt.shape[0]
        y = jnp.dot(cur_ref[:, 0:k_in], wt[...],
                    preferred_element_type=jnp.float32) + b[...]
        o = SEG_OFF[lvl + 1]
        ry = jnp.maximum(y, 0.0)
        if o + SEG_W[lvl + 1] <= cur_ref.shape[1]:
            cur_ref[:, o:o + SEG_W[lvl + 1]] = ry
        out_ref[:, o:o + SEG_W[lvl + 1]] = ry.astype(bf16)


def _scatter_kernel(lab_ref, wsrc_ref, out_ref, stage_a, stage_b, sem_a,
                    sem_b, sem_out):
    step = pl.program_id(0)
    nsteps = pl.num_programs(0)
    rows = stage_a.shape[0]
    base = step * rows

    @pl.when(step == 0)
    def _():
        pltpu.make_async_copy(
            wsrc_ref.at[pl.ds(0, rows)], stage_a, sem_a).start()

    def body(cur_stage, cur_sem, nxt_stage, nxt_sem):
        pltpu.make_async_copy(
            wsrc_ref.at[pl.ds(base, rows)], cur_stage, cur_sem).wait()
        @pl.when(step >= 1)
        def _():
            pltpu.make_async_copy(
                nxt_stage.at[pl.ds(0, rows)],
                out_ref.at[pl.ds(0, rows)], sem_out).wait()

        @pl.when(step + 1 < nsteps)
        def _():
            pltpu.make_async_copy(
                wsrc_ref.at[pl.ds(base + rows, rows)], nxt_stage,
                nxt_sem).start()

        def issue(i, _):
            dst = lab_ref[base + i]
            pltpu.make_async_copy(
                cur_stage.at[i], out_ref.at[dst], sem_out).start()
            return 0

        lax.fori_loop(0, rows, issue, 0)
        @pl.when(step == nsteps - 1)
        def _():
            pltpu.make_async_copy(
                cur_stage.at[pl.ds(0, rows)],
                out_ref.at[pl.ds(0, rows)], sem_out).wait()

    @pl.when(step % 2 == 0)
    def _():
        body(stage_a, sem_a, stage_b, sem_b)

    @pl.when(step % 2 == 1)
    def _():
        body(stage_b, sem_b, stage_a, sem_a)


def _matmul_kernel(cur_ref, w_ref, out_ref):
    out_ref[...] = lax.dot_general(
        cur_ref[...], w_ref[...],
        dimension_numbers=(((1,), (1,)), ((), ())),
        preferred_element_type=jnp.float32)


def kernel(x, W0, b0, W1, b1, W2, b2, W3, b3, W4, b4, W5, b5, labels):
    batch = x.shape[0]
    f32 = jnp.float32
    Ws = [W0, W1, W2, W3, W4, W5]
    bs = [b0, b1, b2, b3, b4, b5]

    wts = [Ws[lvl].T for lvl in range(5)]
    brs = [b.reshape(1, -1) for b in bs]

    blocks = []
    lvl0 = jnp.concatenate([
        jnp.zeros((6, Y0_OFF), f32),
        jnp.eye(6, dtype=f32),
        jnp.zeros((6, K_AUG - Y0_OFF - 6), f32),
    ], axis=1)
    blocks.append(lvl0)
    for lvl in range(1, 6):
        w = Ws[lvl]
        n = w.shape[0]
        blocks.append(jnp.concatenate([
            w,
            jnp.zeros((n, ONES_COL - w.shape[1]), f32),
            bs[lvl].reshape(n, 1),
            jnp.zeros((n, K_AUG - ONES_COL - 1), f32),
        ], axis=1))
    blocks.append(jnp.zeros((2, K_AUG), f32))
    wsrc = jnp.concatenate(blocks, axis=0)

    labels_i = labels.astype(jnp.int32)
    lab_ext = jnp.concatenate(
        [labels_i, jnp.array([NUM_LABELS_K, NUM_LABELS_K + 1], jnp.int32)])

    bb = 256
    grid1 = (batch // bb,)
    cur = pl.pallas_call(
        _phase_a_kernel,
        grid=grid1,
        in_specs=[pl.BlockSpec((bb, IN_FEAT), lambda i: (i, 0))] + [
            spec for lvl in range(5) for spec in (
                pl.BlockSpec(wts[lvl].shape, lambda i: (0, 0)),
                pl.BlockSpec(brs[lvl].shape, lambda i: (0, 0)),
            )
        ],
        out_specs=pl.BlockSpec((bb, K_AUG), lambda i: (i, 0)),
        out_shape=jax.ShapeDtypeStruct((batch, K_AUG), jnp.bfloat16),
        scratch_shapes=[pltpu.VMEM((bb, SEG_OFF[5]), f32)],
        compiler_params=pltpu.CompilerParams(
            dimension_semantics=("parallel",),
            vmem_limit_bytes=56 * 1024 * 1024,
        ),
    )(x, wts[0], brs[0], wts[1], brs[1], wts[2], brs[2], wts[3], brs[3],
      wts[4], brs[4])

    rows_blk = 512
    wsrc3 = wsrc.reshape(N_PAD, 1, K_AUG)
    wbig3 = pl.pallas_call(
        _scatter_kernel,
        grid=(N_PAD // rows_blk,),
        in_specs=[
            pl.BlockSpec(memory_space=pltpu.SMEM),
            pl.BlockSpec(memory_space=pl.ANY),
        ],
        out_specs=pl.BlockSpec(memory_space=pl.ANY),
        out_shape=jax.ShapeDtypeStruct((N_PAD, 1, K_AUG), f32),
        scratch_shapes=[
            pltpu.VMEM((rows_blk, 1, K_AUG), f32),
            pltpu.VMEM((rows_blk, 1, K_AUG), f32),
            pltpu.SemaphoreType.DMA,
            pltpu.SemaphoreType.DMA,
            pltpu.SemaphoreType.DMA,
        ],
        compiler_params=pltpu.CompilerParams(
            dimension_semantics=("arbitrary",),
        ),
    )(lab_ext, wsrc3)

    wbig16 = wbig3.reshape(N_PAD, K_AUG).astype(jnp.bfloat16)
    bm, bn = 1024, 2048
    grid3 = (N_PAD // bn, batch // bm)
    logits = pl.pallas_call(
        _matmul_kernel,
        grid=grid3,
        in_specs=[
            pl.BlockSpec((bm, K_AUG), lambda c, b: (b, 0)),
            pl.BlockSpec((bn, K_AUG), lambda c, b: (c, 0)),
        ],
        out_specs=pl.BlockSpec((bm, bn), lambda c, b: (b, c)),
        out_shape=jax.ShapeDtypeStruct((batch, NUM_LABELS_K), f32),
        compiler_params=pltpu.CompilerParams(
            dimension_semantics=("parallel", "arbitrary"),
            vmem_limit_bytes=56 * 1024 * 1024,
        ),
    )(cur, wbig16)

    return logits


# speedup vs baseline: 1.1132x; 1.0065x over previous
import jax
import jax.numpy as jnp
from jax import lax
from jax.experimental import pallas as pl
from jax.experimental.pallas import tpu as pltpu

LEVEL_SIZES_K = [6, 24, 96, 384, 1536, 6144]
NUM_LABELS_K = 8190
IN_FEAT = 768
K_AUG = 2944
N_PAD = 8192
SEG_OFF = [0, 768, 774, 798, 894, 1278]
SEG_W = [768, 6, 24, 96, 384, 1536]
ONES_COL = 2814
Y0_OFF = 2815


def _phase_a_kernel(x_ref, w0t, b0, w1t, b1, w2t, b2, w3t, b3, w4t, b4,
                    out_ref, cur_ref):
    bb = x_ref.shape[0]
    bf16 = jnp.bfloat16
    out_ref[...] = jnp.zeros((bb, K_AUG), bf16)
    cur_ref[...] = jnp.zeros(cur_ref.shape, jnp.float32)
    x = x_ref[...]
    rx = jnp.maximum(x, 0.0)
    cur_ref[:, 0:768] = rx
    out_ref[:, 0:768] = rx.astype(bf16)
    y0 = jnp.dot(x, w0t[...], preferred_element_type=jnp.float32) + b0[...]
    ry0 = jnp.maximum(y0, 0.0)
    cur_ref[:, 768:774] = ry0
    out_ref[:, 768:774] = ry0.astype(bf16)
    out_ref[:, Y0_OFF:Y0_OFF + 6] = y0.astype(bf16)
    out_ref[:, ONES_COL:ONES_COL + 1] = jnp.ones((bb, 1), bf16)
    for lvl, (wt, b) in enumerate(((w1t, b1), (w2t, b2), (w3t, b3),
                                   (w4t, b4)), start=1):
        k_in = wt.shape[0]
        y = jnp.dot(cur_ref[:, 0:k_in], wt[...],
                    preferred_element_type=jnp.float32) + b[...]
        o = SEG_OFF[lvl + 1]
        ry = jnp.maximum(y, 0.0)
        if o + SEG_W[lvl + 1] <= cur_ref.shape[1]:
            cur_ref[:, o:o + SEG_W[lvl + 1]] = ry
        out_ref[:, o:o + SEG_W[lvl + 1]] = ry.astype(bf16)


def _scatter_kernel(lab_ref, wsrc_ref, out_ref, stage_a, stage_b, sem_a,
                    sem_b, sem_out):
    step = pl.program_id(0)
    nsteps = pl.num_programs(0)
    rows = stage_a.shape[0]
    base = step * rows

    @pl.when(step == 0)
    def _():
        pltpu.make_async_copy(
            wsrc_ref.at[pl.ds(0, rows)], stage_a, sem_a).start()

    def body(cur_stage, cur_sem, nxt_stage, nxt_sem):
        pltpu.make_async_copy(
            wsrc_ref.at[pl.ds(base, rows)], cur_stage, cur_sem).wait()
        @pl.when(step >= 1)
        def _():
            pltpu.make_async_copy(
                nxt_stage.at[pl.ds(0, rows)],
                out_ref.at[pl.ds(0, rows)], sem_out).wait()

        @pl.when(step + 1 < nsteps)
        def _():
            pltpu.make_async_copy(
                wsrc_ref.at[pl.ds(base + rows, rows)], nxt_stage,
                nxt_sem).start()

        def issue(i, _):
            dst = lab_ref[base + i]
            pltpu.make_async_copy(
                cur_stage.at[i], out_ref.at[dst], sem_out).start()
            return 0

        lax.fori_loop(0, rows, issue, 0)
        @pl.when(step == nsteps - 1)
        def _():
            pltpu.make_async_copy(
                cur_stage.at[pl.ds(0, rows)],
                out_ref.at[pl.ds(0, rows)], sem_out).wait()

    @pl.when(step % 2 == 0)
    def _():
        body(stage_a, sem_a, stage_b, sem_b)

    @pl.when(step % 2 == 1)
    def _():
        body(stage_b, sem_b, stage_a, sem_a)


def _matmul_kernel(cur_ref, w_ref, out_ref):
    out_ref[...] = lax.dot_general(
        cur_ref[...], w_ref[...],
        dimension_numbers=(((1,), (1,)), ((), ())),
        preferred_element_type=jnp.float32)


def kernel(x, W0, b0, W1, b1, W2, b2, W3, b3, W4, b4, W5, b5, labels):
    batch = x.shape[0]
    f32 = jnp.float32
    Ws = [W0, W1, W2, W3, W4, W5]
    bs = [b0, b1, b2, b3, b4, b5]

    wts = [Ws[lvl].T for lvl in range(5)]
    brs = [b.reshape(1, -1) for b in bs]

    blocks = []
    lvl0 = jnp.concatenate([
        jnp.zeros((6, Y0_OFF), f32),
        jnp.eye(6, dtype=f32),
        jnp.zeros((6, K_AUG - Y0_OFF - 6), f32),
    ], axis=1)
    blocks.append(lvl0)
    for lvl in range(1, 6):
        w = Ws[lvl]
        n = w.shape[0]
        blocks.append(jnp.concatenate([
            w,
            jnp.zeros((n, ONES_COL - w.shape[1]), f32),
            bs[lvl].reshape(n, 1),
            jnp.zeros((n, K_AUG - ONES_COL - 1), f32),
        ], axis=1))
    blocks.append(jnp.zeros((2, K_AUG), f32))
    wsrc = jnp.concatenate(blocks, axis=0)

    labels_i = labels.astype(jnp.int32)
    lab_ext = jnp.concatenate(
        [labels_i, jnp.array([NUM_LABELS_K, NUM_LABELS_K + 1], jnp.int32)])

    bb = 256
    grid1 = (batch // bb,)
    cur = pl.pallas_call(
        _phase_a_kernel,
        grid=grid1,
        in_specs=[pl.BlockSpec((bb, IN_FEAT), lambda i: (i, 0))] + [
            spec for lvl in range(5) for spec in (
                pl.BlockSpec(wts[lvl].shape, lambda i: (0, 0)),
                pl.BlockSpec(brs[lvl].shape, lambda i: (0, 0)),
            )
        ],
        out_specs=pl.BlockSpec((bb, K_AUG), lambda i: (i, 0)),
        out_shape=jax.ShapeDtypeStruct((batch, K_AUG), jnp.bfloat16),
        scratch_shapes=[pltpu.VMEM((bb, SEG_OFF[5]), f32)],
        compiler_params=pltpu.CompilerParams(
            dimension_semantics=("parallel",),
            vmem_limit_bytes=56 * 1024 * 1024,
        ),
    )(x, wts[0], brs[0], wts[1], brs[1], wts[2], brs[2], wts[3], brs[3],
      wts[4], brs[4])

    rows_blk = 1024
    wsrc3 = wsrc.reshape(N_PAD, 1, K_AUG)
    wbig3 = pl.pallas_call(
        _scatter_kernel,
        grid=(N_PAD // rows_blk,),
        in_specs=[
            pl.BlockSpec(memory_space=pltpu.SMEM),
            pl.BlockSpec(memory_space=pl.ANY),
        ],
        out_specs=pl.BlockSpec(memory_space=pl.ANY),
        out_shape=jax.ShapeDtypeStruct((N_PAD, 1, K_AUG), f32),
        scratch_shapes=[
            pltpu.VMEM((rows_blk, 1, K_AUG), f32),
            pltpu.VMEM((rows_blk, 1, K_AUG), f32),
            pltpu.SemaphoreType.DMA,
            pltpu.SemaphoreType.DMA,
            pltpu.SemaphoreType.DMA,
        ],
        compiler_params=pltpu.CompilerParams(
            dimension_semantics=("arbitrary",),
        ),
    )(lab_ext, wsrc3)

    wbig16 = wbig3.reshape(N_PAD, K_AUG).astype(jnp.bfloat16)
    bm, bn = 1024, 2048
    grid3 = (N_PAD // bn, batch // bm)
    logits = pl.pallas_call(
        _matmul_kernel,
        grid=grid3,
        in_specs=[
            pl.BlockSpec((bm, K_AUG), lambda c, b: (b, 0)),
            pl.BlockSpec((bn, K_AUG), lambda c, b: (c, 0)),
        ],
        out_specs=pl.BlockSpec((bm, bn), lambda c, b: (b, c)),
        out_shape=jax.ShapeDtypeStruct((batch, NUM_LABELS_K), f32),
        compiler_params=pltpu.CompilerParams(
            dimension_semantics=("parallel", "arbitrary"),
            vmem_limit_bytes=56 * 1024 * 1024,
        ),
    )(cur, wbig16)

    return logits
